# baseline (speedup 1.0000x reference)
import functools

import jax
import jax.numpy as jnp
from jax import lax
from jax.experimental import pallas as pl
from jax.experimental.pallas import tpu as pltpu
from jax.experimental.pallas import tpu_sc as plsc

_B = 16
_L = 200
_DS = 64
_N = 10000
_E = 320000
_D = 128
_N2 = 10240
_EPAD = 327680
_CH = 128
_TILES = 16
_RPT = _N2 // _TILES
_ZROWS = _RPT
_BLK = 2048
_GRID = _N2 // _BLK
_NT = _N2 + 2048
_GRIDT = _NT // _BLK
_SB = 8
_NCH = _EPAD // 32 // _CH
_NSB = _NCH // _SB
_NBUF = 4

_mesh = plsc.VectorSubcoreMesh(core_axis_name="c", subcore_axis_name="s")


def _sc_deg_body(dstm_hbm, deg_hbm, blk0, blk1, ones_v, init_v, acc_sh,
                 isem, s0, s1):
    c = lax.axis_index("c")
    s = lax.axis_index("s")
    val = jnp.where(c == 0, 1.0, 0.0).astype(jnp.float32)
    vec16 = jnp.zeros((16,), jnp.float32) + val
    one16 = jnp.zeros((16,), jnp.float32) + 1.0
    for i in range(_CH // 16):
        ones_v[pl.ds(i * 16, 16)] = one16
    for i in range(_RPT // 16):
        init_v[pl.ds(i * 16, 16)] = vec16
    pltpu.sync_copy(init_v, acc_sh.at[pl.ds(s * _RPT, _RPT)])
    plsc.subcore_barrier()

    blks = (blk0, blk1)
    ss = (s0, s1)
    cbase = (c * _TILES + s) * _NCH

    def didx(j):
        sb, jj = divmod(j, _SB)
        return blks[sb % 2].at[jj]

    def start_scatter(j):
        pltpu.async_copy(ones_v, acc_sh.at[didx(j)], ss[j % 2], add=True)

    def wait_scatter(j):
        pltpu.make_async_copy(ones_v, acc_sh.at[didx(j)], ss[j % 2]).wait()

    pltpu.sync_copy(dstm_hbm.at[pl.ds(cbase, _SB)], blk0)
    pltpu.async_copy(dstm_hbm.at[pl.ds(cbase + _SB, _SB)], blk1, isem)
    for j in range(_NCH):
        sb, jj = divmod(j, _SB)
        if jj == 4 and 1 <= sb < _NSB - 1:
            pltpu.async_copy(dstm_hbm.at[pl.ds(cbase + (sb + 1) * _SB, _SB)],
                             blks[(sb + 1) % 2], isem)
        if jj == 0 and sb > 0:
            pltpu.make_async_copy(dstm_hbm.at[pl.ds(cbase + sb * _SB, _SB)],
                                  blks[sb % 2], isem).wait()
        if j >= 2:
            wait_scatter(j - 2)
        start_scatter(j)
    wait_scatter(_NCH - 2)
    wait_scatter(_NCH - 1)

    plsc.subcore_barrier()
    pltpu.sync_copy(acc_sh.at[pl.ds(s * _RPT, _RPT)],
                    deg_hbm.at[pl.ds(c * _N2 + s * _RPT, _RPT)])


_sc_deg = pl.kernel(
    _sc_deg_body,
    out_type=jax.ShapeDtypeStruct((2 * _N2,), jnp.float32),
    mesh=_mesh,
    compiler_params=pltpu.CompilerParams(use_tc_tiling_on_sc=False),
    scratch_types=[
        pltpu.VMEM((_SB, _CH), jnp.int32),
        pltpu.VMEM((_SB, _CH), jnp.int32),
        pltpu.VMEM((_CH,), jnp.float32),
        pltpu.VMEM((_RPT,), jnp.float32),
        pltpu.VMEM_SHARED((_N2,), jnp.float32),
        pltpu.SemaphoreType.DMA,
        pltpu.SemaphoreType.DMA,
        pltpu.SemaphoreType.DMA,
    ],
)


def _sc_edge_body(ytab_hbm, srcm_hbm, dstm_hbm, acc_hbm,
                  sblk0, sblk1, dblk0, dblk1, rows0, rows1, acc_sh,
                  isem, g0, g1, s0, s1):
    c = lax.axis_index("c")
    s = lax.axis_index("s")
    init_off = jnp.where(c == 0, s * _RPT, _N2)
    pltpu.sync_copy(ytab_hbm.at[pl.ds(init_off, _RPT)],
                    acc_sh.at[pl.ds(s * _RPT, _RPT)])
    plsc.subcore_barrier()

    sblks = (sblk0, sblk1)
    dblks = (dblk0, dblk1)
    rows = (rows0, rows1)
    gs = (g0, g1)
    ss = (s0, s1)
    cbase = (c * _TILES + s) * _NCH

    def sidx(j):
        sb, jj = divmod(j, _SB)
        return sblks[sb % 2].at[jj]

    def didx(j):
        sb, jj = divmod(j, _SB)
        return dblks[sb % 2].at[jj]

    def start_gather(j):
        pltpu.async_copy(ytab_hbm.at[sidx(j)], rows[j % 2], gs[j % 2])

    def wait_gather(j):
        pltpu.make_async_copy(ytab_hbm.at[sidx(j)], rows[j % 2],
                              gs[j % 2]).wait()

    def start_scatter(j):
        pltpu.async_copy(rows[j % 2], acc_sh.at[didx(j)],
                         ss[j % 2], add=True)

    def wait_scatter(j):
        pltpu.make_async_copy(rows[j % 2], acc_sh.at[didx(j)],
                              ss[j % 2]).wait()

    def load_sb(sb, sync):
        if sync:
            pltpu.sync_copy(srcm_hbm.at[pl.ds(cbase + sb * _SB, _SB)],
                            sblks[sb % 2])
            pltpu.sync_copy(dstm_hbm.at[pl.ds(cbase + sb * _SB, _SB)],
                            dblks[sb % 2])
        else:
            pltpu.async_copy(srcm_hbm.at[pl.ds(cbase + sb * _SB, _SB)],
                             sblks[sb % 2], isem)
            pltpu.async_copy(dstm_hbm.at[pl.ds(cbase + sb * _SB, _SB)],
                             dblks[sb % 2], isem)

    def wait_sb(sb):
        pltpu.make_async_copy(srcm_hbm.at[pl.ds(cbase + sb * _SB, _SB)],
                              sblks[sb % 2], isem).wait()
        pltpu.make_async_copy(dstm_hbm.at[pl.ds(cbase + sb * _SB, _SB)],
                              dblks[sb % 2], isem).wait()

    load_sb(0, True)
    load_sb(1, False)
    start_gather(0)
    for j in range(_NCH):
        sb, jj = divmod(j, _SB)
        if jj == 4 and 1 <= sb < _NSB - 1:
            load_sb(sb + 1, False)
        if j + 1 < _NCH:
            sbn, jjn = divmod(j + 1, _SB)
            if jjn == 0:
                wait_sb(sbn)
            if j >= 1:
                wait_scatter(j - 1)
            start_gather(j + 1)
        wait_gather(j)
        start_scatter(j)
    wait_scatter(_NCH - 2)
    wait_scatter(_NCH - 1)

    plsc.subcore_barrier()
    pltpu.sync_copy(acc_sh.at[pl.ds(s * _RPT, _RPT)],
                    acc_hbm.at[pl.ds(c * _N2 + s * _RPT, _RPT)])


_sc_edge = pl.kernel(
    _sc_edge_body,
    out_type=jax.ShapeDtypeStruct((2 * _N2, _D), jnp.float32),
    mesh=_mesh,
    scratch_types=[
        pltpu.VMEM((_SB, _CH), jnp.int32),
        pltpu.VMEM((_SB, _CH), jnp.int32),
        pltpu.VMEM((_SB, _CH), jnp.int32),
        pltpu.VMEM((_SB, _CH), jnp.int32),
        pltpu.VMEM((_CH, _D), jnp.float32),
        pltpu.VMEM((_CH, _D), jnp.float32),
        pltpu.VMEM_SHARED((_N2, _D), jnp.float32),
        pltpu.SemaphoreType.DMA,
        pltpu.SemaphoreType.DMA,
        pltpu.SemaphoreType.DMA,
        pltpu.SemaphoreType.DMA,
        pltpu.SemaphoreType.DMA,
    ],
)


def _t0_body(x_ref, deg_ref, w1_ref, y_ref, dinv_ref):
    ones21 = jnp.ones((2, 1), jnp.float32)
    deg = lax.dot_general(deg_ref[...], ones21,
                          (((0,), (0,)), ((), ())),
                          preferred_element_type=jnp.float32)
    dinv = lax.rsqrt(jnp.maximum(deg, 1.0))
    xw = jnp.dot(x_ref[...], w1_ref[...], preferred_element_type=jnp.float32)
    y_ref[...] = xw * dinv
    dinv_ref[...] = dinv


def _t0(x_p, deg_p, w1):
    return pl.pallas_call(
        _t0_body,
        grid=(_GRIDT,),
        in_specs=[
            pl.BlockSpec((_BLK, _D), lambda i: (i, 0)),
            pl.BlockSpec((2, _BLK), lambda i: (0, i)),
            pl.BlockSpec((_D, _D), lambda i: (0, 0)),
        ],
        out_specs=[
            pl.BlockSpec((_BLK, _D), lambda i: (i, 0)),
            pl.BlockSpec((_BLK, 1), lambda i: (i, 0)),
        ],
        out_shape=[
            jax.ShapeDtypeStruct((_NT, _D), jnp.float32),
            jax.ShapeDtypeStruct((_NT, 1), jnp.float32),
        ],
    )(x_p, deg_p, w1)


def _t1_body(acc_ref, dinv_ref, w2_ref, b1_ref, y2_ref):
    i = pl.program_id(0)
    acc = acc_ref[0] + acc_ref[1]
    dinv = dinv_ref[...]
    h1 = jnp.maximum(acc * dinv + b1_ref[...], 0.0)
    xw2 = jnp.dot(h1, w2_ref[...], preferred_element_type=jnp.float32)
    y2_ref[...] = jnp.where(i < _GRID, xw2 * dinv, 0.0)


def _t1(acc1, dinv, w2, b1r):
    return pl.pallas_call(
        _t1_body,
        grid=(_GRIDT,),
        in_specs=[
            pl.BlockSpec((2, _BLK, _D),
                         lambda i: (0, jnp.minimum(i, _GRID - 1), 0)),
            pl.BlockSpec((_BLK, 1), lambda i: (i, 0)),
            pl.BlockSpec((_D, _D), lambda i: (0, 0)),
            pl.BlockSpec((1, _D), lambda i: (0, 0)),
        ],
        out_specs=pl.BlockSpec((_BLK, _D), lambda i: (i, 0)),
        out_shape=jax.ShapeDtypeStruct((_NT, _D), jnp.float32),
    )(acc1, dinv, w2, b1r)


def _t2_body(acc_ref, dinv_ref, batch_ref, b2_ref, hl_ref,
             wc1_ref, bc1_ref, wc2_ref, bc2_ref, out_ref, sums_s, cnt_s):
    i = pl.program_id(0)

    @pl.when(i == 0)
    def _init():
        sums_s[...] = jnp.zeros_like(sums_s)
        cnt_s[...] = jnp.zeros_like(cnt_s)

    acc = acc_ref[0] + acc_ref[1]
    h2 = jnp.maximum(acc * dinv_ref[...] + b2_ref[...], 0.0)
    rows = lax.broadcasted_iota(jnp.int32, (_B, _BLK), 0)
    mask = (rows == batch_ref[...]).astype(jnp.float32)
    sums_s[...] += jnp.dot(mask, h2, preferred_element_type=jnp.float32)
    cnt_s[...] += jnp.broadcast_to(
        jnp.sum(mask, axis=1, keepdims=True), (_B, _D))

    @pl.when(i == _GRID - 1)
    def _fin():
        hg = sums_s[...] / jnp.maximum(cnt_s[...], 1.0)
        fused = jnp.concatenate([hl_ref[...], hg], axis=1)
        z = jnp.maximum(
            jnp.dot(fused, wc1_ref[...], preferred_element_type=jnp.float32)
            + bc1_ref[...], 0.0)
        out_ref[...] = (
            jnp.dot(z, wc2_ref[...], preferred_element_type=jnp.float32)
            + bc2_ref[...])


def _t2(acc2, dinv, batch_r, b2r, h_lstm, wc1, bc1r, wc2p, bc2p):
    return pl.pallas_call(
        _t2_body,
        grid=(_GRID,),
        in_specs=[
            pl.BlockSpec((2, _BLK, _D), lambda i: (0, i, 0)),
            pl.BlockSpec((_BLK, 1), lambda i: (i, 0)),
            pl.BlockSpec((1, _BLK), lambda i: (0, i)),
            pl.BlockSpec((1, _D), lambda i: (0, 0)),
            pl.BlockSpec((_B, _D), lambda i: (0, 0)),
            pl.BlockSpec((2 * _D, _D), lambda i: (0, 0)),
            pl.BlockSpec((1, _D), lambda i: (0, 0)),
            pl.BlockSpec((_D, _D), lambda i: (0, 0)),
            pl.BlockSpec((1, _D), lambda i: (0, 0)),
        ],
        out_specs=pl.BlockSpec((_B, _D), lambda i: (0, 0)),
        out_shape=jax.ShapeDtypeStruct((_B, _D), jnp.float32),
        scratch_shapes=[
            pltpu.VMEM((_B, _D), jnp.float32),
            pltpu.VMEM((_B, _D), jnp.float32),
        ],
    )(acc2, dinv, batch_r, b2r, h_lstm, wc1, bc1r, wc2p, bc2p)


def _lstm_body(seqs_ref, lens_ref, wx_ref, wh_ref, b_ref, out_ref):
    lens = jnp.clip(lens_ref[...], 1, _L)
    wx = wx_ref[...]
    wh = wh_ref[...]
    b = b_ref[...]

    def step(t, carry):
        h, c, hl = carry
        x_t = seqs_ref[:, pl.ds(t, 1), :].reshape(_B, _DS)
        gates = (jnp.dot(x_t, wx, preferred_element_type=jnp.float32) + b
                 + jnp.dot(h, wh, preferred_element_type=jnp.float32))
        ii = jax.nn.sigmoid(gates[:, 0 * _D:1 * _D])
        ff = jax.nn.sigmoid(gates[:, 1 * _D:2 * _D])
        gg = jnp.tanh(gates[:, 2 * _D:3 * _D])
        oo = jax.nn.sigmoid(gates[:, 3 * _D:4 * _D])
        c = ff * c + ii * gg
        h = oo * jnp.tanh(c)
        hl = jnp.where(lens == t + 1, h, hl)
        return h, c, hl

    z = jnp.zeros((_B, _D), jnp.float32)
    _, _, hl = lax.fori_loop(0, _L, step, (z, z, z))
    out_ref[...] = hl


def _lstm(seqs_f, lens_r, wx, wh, br):
    return pl.pallas_call(
        _lstm_body,
        out_shape=jax.ShapeDtypeStruct((_B, _D), jnp.float32),
    )(seqs_f, lens_r, wx, wh, br)


def kernel(seqs, seq_lens, x, edge_index, batch,
           W_ih, W_hh, b_ih, b_hh, W1, b1, W2, b2, Wc1, bc1, Wc2, bc2):
    f32 = jnp.float32
    src = edge_index[0].astype(jnp.int32)
    dst = edge_index[1].astype(jnp.int32)
    pad_e = _EPAD - _E
    spread = jnp.arange(pad_e, dtype=jnp.int32)
    pad_src = _N2 + spread % (_NT - _N2)
    pad_dst = _N + spread % (_N2 - _N)
    dst_p = jnp.concatenate([dst, pad_dst])
    src_p = jnp.concatenate([src, pad_src])
    srcm = src_p.reshape(-1, _CH)
    dstm = dst_p.reshape(-1, _CH)

    x_p = jnp.pad(x.astype(f32), ((0, _NT - _N), (0, 0)))
    batch_r = jnp.pad(batch.astype(jnp.int32), (0, _N2 - _N),
                      constant_values=255).reshape(1, _N2)
    lens_r = seq_lens.astype(jnp.int32).reshape(_B, 1)
    seqs_f = seqs.astype(f32)
    wx = W_ih.astype(f32).T
    wh = W_hh.astype(f32).T
    br = (b_ih + b_hh).astype(f32).reshape(1, 4 * _D)
    b1r = b1.astype(f32).reshape(1, _D)
    b2r = b2.astype(f32).reshape(1, _D)
    bc1r = bc1.astype(f32).reshape(1, _D)
    wc2p = jnp.zeros((_D, _D), f32).at[:, :2].set(Wc2.astype(f32))
    bc2p = jnp.zeros((1, _D), f32).at[0, :2].set(bc2.astype(f32))

    h_lstm = _lstm(seqs_f, lens_r, wx, wh, br)

    deg2 = _sc_deg(dstm)
    deg_p = jnp.pad(deg2.reshape(2, _N2), ((0, 0), (0, _NT - _N2)),
                    constant_values=1.0)
    y1, dinv = _t0(x_p, deg_p, W1.astype(f32))
    acc1 = _sc_edge(y1, srcm, dstm)
    y2 = _t1(acc1.reshape(2, _N2, _D), dinv, W2.astype(f32), b1r)
    acc2 = _sc_edge(y2, srcm, dstm)
    out_p = _t2(acc2.reshape(2, _N2, _D), dinv, batch_r, b2r,
                h_lstm, Wc1.astype(f32), bc1r, wc2p, bc2p)
    return out_p[:, :2]


# speedup vs baseline: 1.0218x; 1.0218x over previous
import functools

import jax
import jax.numpy as jnp
from jax import lax
from jax.experimental import pallas as pl
from jax.experimental.pallas import tpu as pltpu
from jax.experimental.pallas import tpu_sc as plsc

_B = 16
_L = 200
_DS = 64
_N = 10000
_E = 320000
_D = 128
_N2 = 10240
_EPAD = 327680
_CH = 128
_TILES = 16
_RPT = _N2 // _TILES
_ZROWS = _RPT
_BLK = 2048
_GRID = _N2 // _BLK
_NT = _N2 + 2048
_GRIDT = _NT // _BLK
_SB = 8
_NCH = _EPAD // 32 // _CH
_NSB = _NCH // _SB
_NBUF = 4

_mesh = plsc.VectorSubcoreMesh(core_axis_name="c", subcore_axis_name="s")


def _sc_deg_body(dstm_hbm, deg_hbm, blk0, blk1, ones_v, init_v, acc_sh,
                 isem, s0, s1):
    c = lax.axis_index("c")
    s = lax.axis_index("s")
    val = jnp.where(c == 0, 1.0, 0.0).astype(jnp.float32)
    vec16 = jnp.zeros((16,), jnp.float32) + val
    one16 = jnp.zeros((16,), jnp.float32) + 1.0
    for i in range(_CH // 16):
        ones_v[pl.ds(i * 16, 16)] = one16
    for i in range(_RPT // 16):
        init_v[pl.ds(i * 16, 16)] = vec16
    pltpu.sync_copy(init_v, acc_sh.at[pl.ds(s * _RPT, _RPT)])
    plsc.subcore_barrier()

    blks = (blk0, blk1)
    ss = (s0, s1)
    cbase = (c * _TILES + s) * _NCH

    def didx(j):
        sb, jj = divmod(j, _SB)
        return blks[sb % 2].at[jj]

    def start_scatter(j):
        pltpu.async_copy(ones_v, acc_sh.at[didx(j)], ss[j % 2], add=True)

    def wait_scatter(j):
        pltpu.make_async_copy(ones_v, acc_sh.at[didx(j)], ss[j % 2]).wait()

    pltpu.sync_copy(dstm_hbm.at[pl.ds(cbase, _SB)], blk0)
    pltpu.async_copy(dstm_hbm.at[pl.ds(cbase + _SB, _SB)], blk1, isem)
    for j in range(_NCH):
        sb, jj = divmod(j, _SB)
        if jj == 4 and 1 <= sb < _NSB - 1:
            pltpu.async_copy(dstm_hbm.at[pl.ds(cbase + (sb + 1) * _SB, _SB)],
                             blks[(sb + 1) % 2], isem)
        if jj == 0 and sb > 0:
            pltpu.make_async_copy(dstm_hbm.at[pl.ds(cbase + sb * _SB, _SB)],
                                  blks[sb % 2], isem).wait()
        if j >= 2:
            wait_scatter(j - 2)
        start_scatter(j)
    wait_scatter(_NCH - 2)
    wait_scatter(_NCH - 1)

    plsc.subcore_barrier()
    pltpu.sync_copy(acc_sh.at[pl.ds(s * _RPT, _RPT)],
                    deg_hbm.at[pl.ds(c * _N2 + s * _RPT, _RPT)])


_sc_deg = pl.kernel(
    _sc_deg_body,
    out_type=jax.ShapeDtypeStruct((2 * _N2,), jnp.float32),
    mesh=_mesh,
    compiler_params=pltpu.CompilerParams(use_tc_tiling_on_sc=False),
    scratch_types=[
        pltpu.VMEM((_SB, _CH), jnp.int32),
        pltpu.VMEM((_SB, _CH), jnp.int32),
        pltpu.VMEM((_CH,), jnp.float32),
        pltpu.VMEM((_RPT,), jnp.float32),
        pltpu.VMEM_SHARED((_N2,), jnp.float32),
        pltpu.SemaphoreType.DMA,
        pltpu.SemaphoreType.DMA,
        pltpu.SemaphoreType.DMA,
    ],
)


def _sc_edge_body(ytab_hbm, srcm_hbm, dstm_hbm, acc_hbm,
                  sblk0, sblk1, dblk0, dblk1, rows0, rows1, acc_sh,
                  isem, g0, g1, s0, s1):
    c = lax.axis_index("c")
    s = lax.axis_index("s")
    sblks = (sblk0, sblk1)
    dblks = (dblk0, dblk1)
    rows = (rows0, rows1)
    gs = (g0, g1)
    ss = (s0, s1)
    cbase = (c * _TILES + s) * _NCH

    def sidx(j):
        sb, jj = divmod(j, _SB)
        return sblks[sb % 2].at[jj]

    def didx(j):
        sb, jj = divmod(j, _SB)
        return dblks[sb % 2].at[jj]

    def start_gather(j):
        pltpu.async_copy(ytab_hbm.at[sidx(j)], rows[j % 2], gs[j % 2])

    def wait_gather(j):
        pltpu.make_async_copy(ytab_hbm.at[sidx(j)], rows[j % 2],
                              gs[j % 2]).wait()

    def start_scatter(j):
        pltpu.async_copy(rows[j % 2], acc_sh.at[didx(j)],
                         ss[j % 2], add=True)

    def wait_scatter(j):
        pltpu.make_async_copy(rows[j % 2], acc_sh.at[didx(j)],
                              ss[j % 2]).wait()

    def load_sb(sb, sync):
        if sync:
            pltpu.sync_copy(srcm_hbm.at[pl.ds(cbase + sb * _SB, _SB)],
                            sblks[sb % 2])
            pltpu.sync_copy(dstm_hbm.at[pl.ds(cbase + sb * _SB, _SB)],
                            dblks[sb % 2])
        else:
            pltpu.async_copy(srcm_hbm.at[pl.ds(cbase + sb * _SB, _SB)],
                             sblks[sb % 2], isem)
            pltpu.async_copy(dstm_hbm.at[pl.ds(cbase + sb * _SB, _SB)],
                             dblks[sb % 2], isem)

    def wait_sb(sb):
        pltpu.make_async_copy(srcm_hbm.at[pl.ds(cbase + sb * _SB, _SB)],
                              sblks[sb % 2], isem).wait()
        pltpu.make_async_copy(dstm_hbm.at[pl.ds(cbase + sb * _SB, _SB)],
                              dblks[sb % 2], isem).wait()

    init_off = jnp.where(c == 0, s * _RPT, _N2)
    icopy = pltpu.async_copy(ytab_hbm.at[pl.ds(init_off, _RPT)],
                             acc_sh.at[pl.ds(s * _RPT, _RPT)], g1)
    load_sb(0, True)
    load_sb(1, False)
    start_gather(0)
    icopy.wait()
    plsc.subcore_barrier()
    for j in range(_NCH):
        sb, jj = divmod(j, _SB)
        if jj == 4 and 1 <= sb < _NSB - 1:
            load_sb(sb + 1, False)
        if j + 1 < _NCH:
            sbn, jjn = divmod(j + 1, _SB)
            if jjn == 0:
                wait_sb(sbn)
            if j >= 1:
                wait_scatter(j - 1)
            start_gather(j + 1)
        wait_gather(j)
        start_scatter(j)
    wait_scatter(_NCH - 2)
    wait_scatter(_NCH - 1)

    plsc.subcore_barrier()
    pltpu.sync_copy(acc_sh.at[pl.ds(s * _RPT, _RPT)],
                    acc_hbm.at[pl.ds(c * _N2 + s * _RPT, _RPT)])


_sc_edge = pl.kernel(
    _sc_edge_body,
    out_type=jax.ShapeDtypeStruct((2 * _N2, _D), jnp.float32),
    mesh=_mesh,
    scratch_types=[
        pltpu.VMEM((_SB, _CH), jnp.int32),
        pltpu.VMEM((_SB, _CH), jnp.int32),
        pltpu.VMEM((_SB, _CH), jnp.int32),
        pltpu.VMEM((_SB, _CH), jnp.int32),
        pltpu.VMEM((_CH, _D), jnp.float32),
        pltpu.VMEM((_CH, _D), jnp.float32),
        pltpu.VMEM_SHARED((_N2, _D), jnp.float32),
        pltpu.SemaphoreType.DMA,
        pltpu.SemaphoreType.DMA,
        pltpu.SemaphoreType.DMA,
        pltpu.SemaphoreType.DMA,
        pltpu.SemaphoreType.DMA,
    ],
)


def _t0_body(x_ref, deg_ref, w1_ref, y_ref, dinv_ref):
    ones21 = jnp.ones((2, 1), jnp.float32)
    deg = lax.dot_general(deg_ref[...], ones21,
                          (((0,), (0,)), ((), ())),
                          preferred_element_type=jnp.float32)
    dinv = lax.rsqrt(jnp.maximum(deg, 1.0))
    xw = jnp.dot(x_ref[...], w1_ref[...], preferred_element_type=jnp.float32)
    y_ref[...] = xw * dinv
    dinv_ref[...] = dinv


def _t0(x_p, deg_p, w1):
    return pl.pallas_call(
        _t0_body,
        grid=(_GRIDT,),
        in_specs=[
            pl.BlockSpec((_BLK, _D), lambda i: (i, 0)),
            pl.BlockSpec((2, _BLK), lambda i: (0, i)),
            pl.BlockSpec((_D, _D), lambda i: (0, 0)),
        ],
        out_specs=[
            pl.BlockSpec((_BLK, _D), lambda i: (i, 0)),
            pl.BlockSpec((_BLK, 1), lambda i: (i, 0)),
        ],
        out_shape=[
            jax.ShapeDtypeStruct((_NT, _D), jnp.float32),
            jax.ShapeDtypeStruct((_NT, 1), jnp.float32),
        ],
    )(x_p, deg_p, w1)


def _t1_body(acc_ref, dinv_ref, w2_ref, b1_ref, y2_ref):
    i = pl.program_id(0)
    acc = acc_ref[0] + acc_ref[1]
    dinv = dinv_ref[...]
    h1 = jnp.maximum(acc * dinv + b1_ref[...], 0.0)
    xw2 = jnp.dot(h1, w2_ref[...], preferred_element_type=jnp.float32)
    y2_ref[...] = jnp.where(i < _GRID, xw2 * dinv, 0.0)


def _t1(acc1, dinv, w2, b1r):
    return pl.pallas_call(
        _t1_body,
        grid=(_GRIDT,),
        in_specs=[
            pl.BlockSpec((2, _BLK, _D),
                         lambda i: (0, jnp.minimum(i, _GRID - 1), 0)),
            pl.BlockSpec((_BLK, 1), lambda i: (i, 0)),
            pl.BlockSpec((_D, _D), lambda i: (0, 0)),
            pl.BlockSpec((1, _D), lambda i: (0, 0)),
        ],
        out_specs=pl.BlockSpec((_BLK, _D), lambda i: (i, 0)),
        out_shape=jax.ShapeDtypeStruct((_NT, _D), jnp.float32),
    )(acc1, dinv, w2, b1r)


def _t2_body(acc_ref, dinv_ref, batch_ref, b2_ref, hl_ref,
             wc1_ref, bc1_ref, wc2_ref, bc2_ref, out_ref, sums_s, cnt_s):
    i = pl.program_id(0)

    @pl.when(i == 0)
    def _init():
        sums_s[...] = jnp.zeros_like(sums_s)
        cnt_s[...] = jnp.zeros_like(cnt_s)

    acc = acc_ref[0] + acc_ref[1]
    h2 = jnp.maximum(acc * dinv_ref[...] + b2_ref[...], 0.0)
    rows = lax.broadcasted_iota(jnp.int32, (_B, _BLK), 0)
    mask = (rows == batch_ref[...]).astype(jnp.float32)
    sums_s[...] += jnp.dot(mask, h2, preferred_element_type=jnp.float32)
    cnt_s[...] += jnp.broadcast_to(
        jnp.sum(mask, axis=1, keepdims=True), (_B, _D))

    @pl.when(i == _GRID - 1)
    def _fin():
        hg = sums_s[...] / jnp.maximum(cnt_s[...], 1.0)
        fused = jnp.concatenate([hl_ref[...], hg], axis=1)
        z = jnp.maximum(
            jnp.dot(fused, wc1_ref[...], preferred_element_type=jnp.float32)
            + bc1_ref[...], 0.0)
        out_ref[...] = (
            jnp.dot(z, wc2_ref[...], preferred_element_type=jnp.float32)
            + bc2_ref[...])


def _t2(acc2, dinv, batch_r, b2r, h_lstm, wc1, bc1r, wc2p, bc2p):
    return pl.pallas_call(
        _t2_body,
        grid=(_GRID,),
        in_specs=[
            pl.BlockSpec((2, _BLK, _D), lambda i: (0, i, 0)),
            pl.BlockSpec((_BLK, 1), lambda i: (i, 0)),
            pl.BlockSpec((1, _BLK), lambda i: (0, i)),
            pl.BlockSpec((1, _D), lambda i: (0, 0)),
            pl.BlockSpec((_B, _D), lambda i: (0, 0)),
            pl.BlockSpec((2 * _D, _D), lambda i: (0, 0)),
            pl.BlockSpec((1, _D), lambda i: (0, 0)),
            pl.BlockSpec((_D, _D), lambda i: (0, 0)),
            pl.BlockSpec((1, _D), lambda i: (0, 0)),
        ],
        out_specs=pl.BlockSpec((_B, _D), lambda i: (0, 0)),
        out_shape=jax.ShapeDtypeStruct((_B, _D), jnp.float32),
        scratch_shapes=[
            pltpu.VMEM((_B, _D), jnp.float32),
            pltpu.VMEM((_B, _D), jnp.float32),
        ],
    )(acc2, dinv, batch_r, b2r, h_lstm, wc1, bc1r, wc2p, bc2p)


def _lstm_body(seqs_ref, lens_ref, wx_ref, wh_ref, b_ref, out_ref):
    lens = jnp.clip(lens_ref[...], 1, _L)
    wx = wx_ref[...]
    wh = wh_ref[...]
    b = b_ref[...]

    def step(t, carry):
        h, c, hl = carry
        x_t = seqs_ref[:, pl.ds(t, 1), :].reshape(_B, _DS)
        gates = (jnp.dot(x_t, wx, preferred_element_type=jnp.float32) + b
                 + jnp.dot(h, wh, preferred_element_type=jnp.float32))
        ii = jax.nn.sigmoid(gates[:, 0 * _D:1 * _D])
        ff = jax.nn.sigmoid(gates[:, 1 * _D:2 * _D])
        gg = jnp.tanh(gates[:, 2 * _D:3 * _D])
        oo = jax.nn.sigmoid(gates[:, 3 * _D:4 * _D])
        c = ff * c + ii * gg
        h = oo * jnp.tanh(c)
        hl = jnp.where(lens == t + 1, h, hl)
        return h, c, hl

    z = jnp.zeros((_B, _D), jnp.float32)
    _, _, hl = lax.fori_loop(0, _L, step, (z, z, z))
    out_ref[...] = hl


def _lstm(seqs_f, lens_r, wx, wh, br):
    return pl.pallas_call(
        _lstm_body,
        out_shape=jax.ShapeDtypeStruct((_B, _D), jnp.float32),
    )(seqs_f, lens_r, wx, wh, br)


def kernel(seqs, seq_lens, x, edge_index, batch,
           W_ih, W_hh, b_ih, b_hh, W1, b1, W2, b2, Wc1, bc1, Wc2, bc2):
    f32 = jnp.float32
    src = edge_index[0].astype(jnp.int32)
    dst = edge_index[1].astype(jnp.int32)
    pad_e = _EPAD - _E
    spread = jnp.arange(pad_e, dtype=jnp.int32)
    pad_src = _N2 + spread % (_NT - _N2)
    pad_dst = _N + spread % (_N2 - _N)
    dst_p = jnp.concatenate([dst, pad_dst])
    src_p = jnp.concatenate([src, pad_src])
    srcm = src_p.reshape(-1, _CH)
    dstm = dst_p.reshape(-1, _CH)

    x_p = jnp.pad(x.astype(f32), ((0, _NT - _N), (0, 0)))
    batch_r = jnp.pad(batch.astype(jnp.int32), (0, _N2 - _N),
                      constant_values=255).reshape(1, _N2)
    lens_r = seq_lens.astype(jnp.int32).reshape(_B, 1)
    seqs_f = seqs.astype(f32)
    wx = W_ih.astype(f32).T
    wh = W_hh.astype(f32).T
    br = (b_ih + b_hh).astype(f32).reshape(1, 4 * _D)
    b1r = b1.astype(f32).reshape(1, _D)
    b2r = b2.astype(f32).reshape(1, _D)
    bc1r = bc1.astype(f32).reshape(1, _D)
    wc2p = jnp.zeros((_D, _D), f32).at[:, :2].set(Wc2.astype(f32))
    bc2p = jnp.zeros((1, _D), f32).at[0, :2].set(bc2.astype(f32))

    h_lstm = _lstm(seqs_f, lens_r, wx, wh, br)

    deg2 = _sc_deg(dstm)
    deg_p = jnp.pad(deg2.reshape(2, _N2), ((0, 0), (0, _NT - _N2)),
                    constant_values=1.0)
    y1, dinv = _t0(x_p, deg_p, W1.astype(f32))
    acc1 = _sc_edge(y1, srcm, dstm)
    y2 = _t1(acc1.reshape(2, _N2, _D), dinv, W2.astype(f32), b1r)
    acc2 = _sc_edge(y2, srcm, dstm)
    out_p = _t2(acc2.reshape(2, _N2, _D), dinv, batch_r, b2r,
                h_lstm, Wc1.astype(f32), bc1r, wc2p, bc2p)
    return out_p[:, :2]
